# Initial kernel scaffold; baseline (speedup 1.0000x reference)
import jax
import jax.numpy as jnp
from jax.experimental import pallas as pl


def kernel(x, adj, W1, W2, W3, W4, W4s, fcW, fcb, bn_gamma, bn_beta, bn_mean, bn_var):
    raise NotImplementedError("write your pallas kernel here")


# speedup vs baseline: 1.1339x; 1.1339x over previous
import functools

import jax
import jax.numpy as jnp
from jax.experimental import pallas as pl
from jax.experimental.pallas import tpu as pltpu

_NEG_SLOPE = 0.01
_BM = 2048
_BK = 2048


def _leaky(t):
    return jnp.where(t >= 0, t, _NEG_SLOPE * t)


def _mask_rows(blk, valid):
    it = jax.lax.broadcasted_iota(jnp.int32, blk.shape, 0)
    return jnp.where(it < valid, blk, 0)


def _mask_cols(blk, valid):
    it = jax.lax.broadcasted_iota(jnp.int32, blk.shape, 1)
    return jnp.where(it < valid, blk, 0)


def _mm_kernel(a_ref, w_ref, o_ref):
    o_ref[...] = jnp.dot(a_ref[...], w_ref[...],
                         preferred_element_type=jnp.float32)


def _small_mm(a, w, bm):
    n, d = a.shape
    fo = w.shape[1]
    return pl.pallas_call(
        _mm_kernel,
        grid=(pl.cdiv(n, bm),),
        in_specs=[pl.BlockSpec((bm, d), lambda i: (i, 0)),
                  pl.BlockSpec((d, fo), lambda i: (0, 0))],
        out_specs=pl.BlockSpec((bm, fo), lambda i: (i, 0)),
        out_shape=jax.ShapeDtypeStruct((n, fo), jnp.float32),
    )(a, w)


def _layer_body(adj_ref, b_ref, acc_ref, *, nk, rem):
    k = pl.program_id(1)

    @pl.when(k == 0)
    def _():
        acc_ref[...] = jnp.zeros_like(acc_ref)

    def _step(mask):
        a = adj_ref[...]
        b = b_ref[...].astype(a.dtype)
        if mask:
            a = _mask_cols(a, rem)
            b = _mask_rows(b, rem)
        acc_ref[...] += jnp.dot(a, b, preferred_element_type=jnp.float32)

    if rem == _BK:
        _step(False)
    else:
        @pl.when(k < nk - 1)
        def _():
            _step(False)

        @pl.when(k == nk - 1)
        def _():
            _step(True)


def _layer_kernel(adj_ref, b_ref, w_ref, o_ref, acc_ref, *, nk, rem):
    _layer_body(adj_ref, b_ref, acc_ref, nk=nk, rem=rem)

    @pl.when(pl.program_id(1) == nk - 1)
    def _():
        h = _leaky(acc_ref[...])
        o_ref[...] = jnp.dot(h, w_ref[...], preferred_element_type=jnp.float32)


def _gcn_layer(adj, b, w_next):
    n = adj.shape[0]
    wd = b.shape[1]
    wo = w_next.shape[1]
    nk = pl.cdiv(n, _BK)
    rem = n - (nk - 1) * _BK
    return pl.pallas_call(
        functools.partial(_layer_kernel, nk=nk, rem=rem),
        grid=(pl.cdiv(n, _BM), nk),
        in_specs=[pl.BlockSpec((_BM, _BK), lambda i, k: (i, k)),
                  pl.BlockSpec((_BK, wd), lambda i, k: (k, 0)),
                  pl.BlockSpec((wd, wo), lambda i, k: (0, 0))],
        out_specs=pl.BlockSpec((_BM, wo), lambda i, k: (i, 0)),
        out_shape=jax.ShapeDtypeStruct((n, wo), jnp.float32),
        scratch_shapes=[pltpu.VMEM((_BM, wd), jnp.float32)],
        compiler_params=pltpu.CompilerParams(
            dimension_semantics=("parallel", "arbitrary")),
    )(adj, b, w_next)


def _final_kernel(adj_ref, b_ref, fcw_ref, fcb_ref, mu_ref, lv_ref, xr_ref,
                  acc_ref, *, nk, rem, h4):
    _layer_body(adj_ref, b_ref, acc_ref, nk=nk, rem=rem)

    @pl.when(pl.program_id(1) == nk - 1)
    def _():
        h = _leaky(acc_ref[...])
        mu = h[:, :h4]
        mu_ref[...] = mu
        lv_ref[...] = h[:, h4:]
        xr_ref[...] = jnp.dot(mu, fcw_ref[...],
                              preferred_element_type=jnp.float32) + fcb_ref[...]


def _gcn_final(adj, b, fcw, fcb):
    n = adj.shape[0]
    wd = b.shape[1]
    h4 = wd // 2
    d = fcw.shape[1]
    nk = pl.cdiv(n, _BK)
    rem = n - (nk - 1) * _BK
    out_shapes = (jax.ShapeDtypeStruct((n, h4), jnp.float32),
                  jax.ShapeDtypeStruct((n, h4), jnp.float32),
                  jax.ShapeDtypeStruct((n, d), jnp.float32))
    return pl.pallas_call(
        functools.partial(_final_kernel, nk=nk, rem=rem, h4=h4),
        grid=(pl.cdiv(n, _BM), nk),
        in_specs=[pl.BlockSpec((_BM, _BK), lambda i, k: (i, k)),
                  pl.BlockSpec((_BK, wd), lambda i, k: (k, 0)),
                  pl.BlockSpec((h4, d), lambda i, k: (0, 0)),
                  pl.BlockSpec((1, d), lambda i, k: (0, 0))],
        out_specs=(pl.BlockSpec((_BM, h4), lambda i, k: (i, 0)),
                   pl.BlockSpec((_BM, h4), lambda i, k: (i, 0)),
                   pl.BlockSpec((_BM, d), lambda i, k: (i, 0))),
        out_shape=out_shapes,
        scratch_shapes=[pltpu.VMEM((_BM, wd), jnp.float32)],
        compiler_params=pltpu.CompilerParams(
            dimension_semantics=("parallel", "arbitrary")),
    )(adj, b, fcw, fcb)


def _dc_kernel(zi_ref, zj_ref, o_ref):
    o_ref[...] = jax.lax.dot_general(
        zi_ref[...], zj_ref[...], (((1,), (1,)), ((), ())),
        preferred_element_type=jnp.float32)


def _decode(z):
    n, h = z.shape
    nb = pl.cdiv(n, _BM)
    return pl.pallas_call(
        _dc_kernel,
        grid=(nb, nb),
        in_specs=[pl.BlockSpec((_BM, h), lambda i, j: (i, 0)),
                  pl.BlockSpec((_BM, h), lambda i, j: (j, 0))],
        out_specs=pl.BlockSpec((_BM, _BM), lambda i, j: (i, j)),
        out_shape=jax.ShapeDtypeStruct((n, n), jnp.float32),
        compiler_params=pltpu.CompilerParams(
            dimension_semantics=("parallel", "parallel")),
    )(z, z)


def kernel(x, adj, W1, W2, W3, W4, W4s, fcW, fcb,
           bn_gamma, bn_beta, bn_mean, bn_var):
    scale = bn_gamma / jnp.sqrt(bn_var + 1e-5)
    fcWp = fcW * scale[None, :]
    fcbp = ((fcb - bn_mean) * scale + bn_beta)[None, :]
    W4c = jnp.concatenate([W4, W4s], axis=1)

    xW1 = _small_mm(x, W1, _BM)
    h1W2 = _gcn_layer(adj, xW1, W2)
    h2W3 = _gcn_layer(adj, h1W2, W3)
    h3W4 = _gcn_layer(adj, h2W3, W4c)
    mu, logvar, xr = _gcn_final(adj, h3W4, fcWp, fcbp)
    dc = _decode(mu)
    return (dc, mu, logvar, mu, xr)


# speedup vs baseline: 1.2693x; 1.1193x over previous
import functools

import jax
import jax.numpy as jnp
from jax.experimental import pallas as pl
from jax.experimental.pallas import tpu as pltpu

_NEG_SLOPE = 0.01
_BM = 2048
_BK = 2048


def _leaky(t):
    return jnp.where(t >= 0, t, _NEG_SLOPE * t)


def _mask_rows(blk, valid):
    it = jax.lax.broadcasted_iota(jnp.int32, blk.shape, 0)
    return jnp.where(it < valid, blk, 0)


def _mask_cols(blk, valid):
    it = jax.lax.broadcasted_iota(jnp.int32, blk.shape, 1)
    return jnp.where(it < valid, blk, 0)


def _mm_kernel(a_ref, w_ref, o_ref):
    o_ref[...] = jnp.dot(a_ref[...], w_ref[...],
                         preferred_element_type=jnp.float32)


def _small_mm(a, w, bm):
    n, d = a.shape
    fo = w.shape[1]
    return pl.pallas_call(
        _mm_kernel,
        grid=(pl.cdiv(n, bm),),
        in_specs=[pl.BlockSpec((bm, d), lambda i: (i, 0)),
                  pl.BlockSpec((d, fo), lambda i: (0, 0))],
        out_specs=pl.BlockSpec((bm, fo), lambda i: (i, 0)),
        out_shape=jax.ShapeDtypeStruct((n, fo), jnp.float32),
    )(a, w)


def _layer_body(adj_ref, b_ref, acc_ref, *, nk, rem):
    k = pl.program_id(1)

    @pl.when(k == 0)
    def _():
        acc_ref[...] = jnp.zeros_like(acc_ref)

    def _step(mask):
        a = adj_ref[...]
        b = b_ref[...].astype(a.dtype)
        if mask:
            a = _mask_cols(a, rem)
            b = _mask_rows(b, rem)
        acc_ref[...] += jnp.dot(a, b, preferred_element_type=jnp.float32)

    if rem == _BK:
        _step(False)
    else:
        @pl.when(k < nk - 1)
        def _():
            _step(False)

        @pl.when(k == nk - 1)
        def _():
            _step(True)


def _layer_kernel(adj_ref, b_ref, w_ref, o_ref, acc_ref, *, nk, rem):
    _layer_body(adj_ref, b_ref, acc_ref, nk=nk, rem=rem)

    @pl.when(pl.program_id(1) == nk - 1)
    def _():
        h = _leaky(acc_ref[...])
        o_ref[...] = jnp.dot(h, w_ref[...], preferred_element_type=jnp.float32)


def _first_layer_kernel(adj_ref, b_ref, w_ref, o_ref, adjb_ref, acc_ref,
                        *, nk, rem, bm1):
    k = pl.program_id(1)

    @pl.when(k == 0)
    def _():
        acc_ref[...] = jnp.zeros_like(acc_ref)

    ab = adj_ref[...].astype(jnp.bfloat16)
    adjb_ref[...] = ab

    def _step(mask):
        a = ab
        b = b_ref[...].astype(jnp.bfloat16)
        if mask:
            a = _mask_cols(a, rem)
            b = _mask_rows(b, rem)
        acc_ref[...] += jnp.dot(a, b, preferred_element_type=jnp.float32)

    if rem == _BK:
        _step(False)
    else:
        @pl.when(k < nk - 1)
        def _():
            _step(False)

        @pl.when(k == nk - 1)
        def _():
            _step(True)

    @pl.when(k == nk - 1)
    def _():
        h = _leaky(acc_ref[...])
        o_ref[...] = jnp.dot(h, w_ref[...], preferred_element_type=jnp.float32)


def _gcn_first_layer(adj, b, w_next, bm1):
    n = adj.shape[0]
    wd = b.shape[1]
    wo = w_next.shape[1]
    nk = pl.cdiv(n, _BK)
    rem = n - (nk - 1) * _BK
    return pl.pallas_call(
        functools.partial(_first_layer_kernel, nk=nk, rem=rem, bm1=bm1),
        grid=(pl.cdiv(n, bm1), nk),
        in_specs=[pl.BlockSpec((bm1, _BK), lambda i, k: (i, k)),
                  pl.BlockSpec((_BK, wd), lambda i, k: (k, 0)),
                  pl.BlockSpec((wd, wo), lambda i, k: (0, 0))],
        out_specs=(pl.BlockSpec((bm1, wo), lambda i, k: (i, 0)),
                   pl.BlockSpec((bm1, _BK), lambda i, k: (i, k))),
        out_shape=(jax.ShapeDtypeStruct((n, wo), jnp.float32),
                   jax.ShapeDtypeStruct((n, n), jnp.bfloat16)),
        scratch_shapes=[pltpu.VMEM((bm1, wd), jnp.float32)],
        compiler_params=pltpu.CompilerParams(
            dimension_semantics=("parallel", "arbitrary")),
    )(adj, b, w_next)


def _gcn_layer(adj, b, w_next):
    n = adj.shape[0]
    wd = b.shape[1]
    wo = w_next.shape[1]
    nk = pl.cdiv(n, _BK)
    rem = n - (nk - 1) * _BK
    return pl.pallas_call(
        functools.partial(_layer_kernel, nk=nk, rem=rem),
        grid=(pl.cdiv(n, _BM), nk),
        in_specs=[pl.BlockSpec((_BM, _BK), lambda i, k: (i, k)),
                  pl.BlockSpec((_BK, wd), lambda i, k: (k, 0)),
                  pl.BlockSpec((wd, wo), lambda i, k: (0, 0))],
        out_specs=pl.BlockSpec((_BM, wo), lambda i, k: (i, 0)),
        out_shape=jax.ShapeDtypeStruct((n, wo), jnp.float32),
        scratch_shapes=[pltpu.VMEM((_BM, wd), jnp.float32)],
        compiler_params=pltpu.CompilerParams(
            dimension_semantics=("parallel", "arbitrary")),
    )(adj, b, w_next)


def _final_kernel(adj_ref, b_ref, fcw_ref, fcb_ref, mu_ref, lv_ref, xr_ref,
                  acc_ref, *, nk, rem, h4):
    _layer_body(adj_ref, b_ref, acc_ref, nk=nk, rem=rem)

    @pl.when(pl.program_id(1) == nk - 1)
    def _():
        h = _leaky(acc_ref[...])
        mu = h[:, :h4]
        mu_ref[...] = mu
        lv_ref[...] = h[:, h4:]
        xr_ref[...] = jnp.dot(mu, fcw_ref[...],
                              preferred_element_type=jnp.float32) + fcb_ref[...]


def _gcn_final(adj, b, fcw, fcb):
    n = adj.shape[0]
    wd = b.shape[1]
    h4 = wd // 2
    d = fcw.shape[1]
    nk = pl.cdiv(n, _BK)
    rem = n - (nk - 1) * _BK
    out_shapes = (jax.ShapeDtypeStruct((n, h4), jnp.float32),
                  jax.ShapeDtypeStruct((n, h4), jnp.float32),
                  jax.ShapeDtypeStruct((n, d), jnp.float32))
    return pl.pallas_call(
        functools.partial(_final_kernel, nk=nk, rem=rem, h4=h4),
        grid=(pl.cdiv(n, _BM), nk),
        in_specs=[pl.BlockSpec((_BM, _BK), lambda i, k: (i, k)),
                  pl.BlockSpec((_BK, wd), lambda i, k: (k, 0)),
                  pl.BlockSpec((h4, d), lambda i, k: (0, 0)),
                  pl.BlockSpec((1, d), lambda i, k: (0, 0))],
        out_specs=(pl.BlockSpec((_BM, h4), lambda i, k: (i, 0)),
                   pl.BlockSpec((_BM, h4), lambda i, k: (i, 0)),
                   pl.BlockSpec((_BM, d), lambda i, k: (i, 0))),
        out_shape=out_shapes,
        scratch_shapes=[pltpu.VMEM((_BM, wd), jnp.float32)],
        compiler_params=pltpu.CompilerParams(
            dimension_semantics=("parallel", "arbitrary")),
    )(adj, b, fcw, fcb)


def _dc_kernel(zi_ref, zj_ref, o_ref):
    o_ref[...] = jax.lax.dot_general(
        zi_ref[...], zj_ref[...], (((1,), (1,)), ((), ())),
        preferred_element_type=jnp.float32)


def _decode(z):
    n, h = z.shape
    nb = pl.cdiv(n, _BM)
    return pl.pallas_call(
        _dc_kernel,
        grid=(nb, nb),
        in_specs=[pl.BlockSpec((_BM, h), lambda i, j: (i, 0)),
                  pl.BlockSpec((_BM, h), lambda i, j: (j, 0))],
        out_specs=pl.BlockSpec((_BM, _BM), lambda i, j: (i, j)),
        out_shape=jax.ShapeDtypeStruct((n, n), jnp.float32),
        compiler_params=pltpu.CompilerParams(
            dimension_semantics=("parallel", "parallel")),
    )(z, z)


def kernel(x, adj, W1, W2, W3, W4, W4s, fcW, fcb,
           bn_gamma, bn_beta, bn_mean, bn_var):
    scale = bn_gamma / jnp.sqrt(bn_var + 1e-5)
    fcWp = fcW * scale[None, :]
    fcbp = ((fcb - bn_mean) * scale + bn_beta)[None, :]
    W4c = jnp.concatenate([W4, W4s], axis=1)

    xW1 = _small_mm(x, W1, _BM)
    h1W2, adj_bf = _gcn_first_layer(adj, xW1, W2, 1024)
    h2W3 = _gcn_layer(adj_bf, h1W2, W3)
    h3W4 = _gcn_layer(adj_bf, h2W3, W4c)
    mu, logvar, xr = _gcn_final(adj_bf, h3W4, fcWp, fcbp)
    dc = _decode(mu.astype(jnp.bfloat16))
    return (dc, mu, logvar, mu, xr)
